# baseline (speedup 1.0000x reference)
import numpy as np
import jax
import jax.numpy as jnp
from jax.experimental import pallas as pl
from jax.experimental.pallas import tpu as pltpu

H = 384
W_IMG = 384
C = 3
P = 16
EMBED = 192
EPAD = 256
N_TARGETS = 4
HP = H // P
WP = W_IMG // P
NPATCH = HP * WP
LB = W_IMG // 128
WB = 128 // P


def _rect_indices(rng, h, w, sfr, arr):
    low_w = int(w * sfr[0])
    high_w = int(w * sfr[1])
    rec_width = int(rng.integers(max(low_w, 1), high_w + 1))
    low_h = int(rec_width * arr[0])
    high_h = int(rec_width * arr[1])
    rec_height = int(rng.integers(max(low_h, 1), high_h + 1))
    rec_height = min(rec_height, h)
    start_w = int(rng.integers(0, w - rec_width + 1))
    start_h = int(rng.integers(0, h - rec_height + 1))
    start = start_h * w + start_w
    idx = np.concatenate(
        [np.arange(start + i * w, start + i * w + rec_width) for i in range(rec_height)]
    )
    return idx.astype(np.int64)


def _mask_indices():
    rng = np.random.default_rng(0)
    targets = [
        _rect_indices(rng, HP, WP, (0.15, 0.2), (0.75, 1.5)) for _ in range(N_TARGETS)
    ]
    ctx = _rect_indices(rng, HP, WP, (0.85, 1.0), (1.0, 1.0))
    all_t = np.concatenate(targets)
    ctx = ctx[~np.isin(ctx, all_t)]
    return ctx, np.concatenate(targets)


CTX_IDX, TGT_IDX = _mask_indices()
N_CTX = len(CTX_IDX)
N_TGT = len(TGT_IDX)


def _runs(idx):
    runs = []
    start = 0
    for i in range(1, len(idx) + 1):
        if i == len(idx) or idx[i] != idx[i - 1] + 1:
            runs.append((int(idx[start]), start, i - start))
            start = i
    return runs


CTX_RUNS = _runs(CTX_IDX)
TGT_RUNS = _runs(TGT_IDX)


BN = 2


def _fused_kernel(x_ref, w4_ref, pm_ref, b_ref, ctx_ref, tgt_ref, wd_ref):
    n = pl.program_id(0)

    @pl.when(n == 0)
    def _init_weights():
        wd_ref[...] = jnp.zeros((P * C * 128, WB * EPAD), jnp.bfloat16)
        for p1 in range(P):
            for c in range(C):
                for w2 in range(WB):
                    r0 = (c * P + p1) * 128 + w2 * P
                    c0 = w2 * EPAD
                    wd_ref[r0 : r0 + P, c0 : c0 + EPAD] = w4_ref[c * P + p1]

    row_blocks = []
    for i in range(BN):
        pieces = []
        for c in range(C):
            xc = x_ref[i, c].reshape(H, W_IMG).astype(jnp.bfloat16)
            y = jax.lax.dot_general(
                pm_ref[...],
                xc,
                (((1,), (0,)), ((), ())),
                preferred_element_type=jnp.float32,
            ).astype(jnp.bfloat16)
            for p1 in range(P):
                blk = y[p1 * HP : (p1 + 1) * HP]
                pieces.append(blk.reshape(HP * LB, 128))
        row_blocks.append(jnp.concatenate(pieces, axis=1))
    lhs = jnp.concatenate(row_blocks, axis=0)
    acc = jax.lax.dot_general(
        lhs,
        wd_ref[...],
        (((1,), (0,)), ((), ())),
        preferred_element_type=jnp.float32,
    )
    for i in range(BN):
        emb = (
            acc[i * HP * LB : (i + 1) * HP * LB].reshape(NPATCH, EPAD)
            + b_ref[...]
        )
        for src, dst, ln in CTX_RUNS:
            ctx_ref[i, dst : dst + ln, :] = emb[src : src + ln, :EMBED]
        for src, dst, ln in TGT_RUNS:
            tgt_ref[i, dst : dst + ln, :] = emb[src : src + ln, :EMBED]


def kernel(x, W, b):
    B = x.shape[0]
    W4 = W.reshape(P, P, C, EMBED).transpose(2, 0, 1, 3).reshape(C * P, P, EMBED)
    W4 = jnp.pad(W4, ((0, 0), (0, 0), (0, EPAD - EMBED))).astype(jnp.bfloat16)
    b2 = jnp.pad(b, (0, EPAD - EMBED)).reshape(1, EPAD)
    pm = np.zeros((H, H), np.float32)
    for p1 in range(P):
        for h in range(HP):
            pm[p1 * HP + h, h * P + p1] = 1.0
    pmat = jnp.asarray(pm, dtype=jnp.bfloat16)

    grid = (B // BN,)
    out = pl.pallas_call(
        _fused_kernel,
        grid=grid,
        in_specs=[
            pl.BlockSpec((BN, C, H, W_IMG), lambda n: (n, 0, 0, 0)),
            pl.BlockSpec((C * P, P, EPAD), lambda n: (0, 0, 0)),
            pl.BlockSpec((H, H), lambda n: (0, 0)),
            pl.BlockSpec((1, EPAD), lambda n: (0, 0)),
        ],
        out_specs=[
            pl.BlockSpec((BN, N_CTX, EMBED), lambda n: (n, 0, 0)),
            pl.BlockSpec((BN, N_TGT, EMBED), lambda n: (n, 0, 0)),
        ],
        out_shape=[
            jax.ShapeDtypeStruct((B, N_CTX, EMBED), jnp.float32),
            jax.ShapeDtypeStruct((B, N_TGT, EMBED), jnp.float32),
        ],
        scratch_shapes=[
            pltpu.VMEM((P * C * 128, WB * EPAD), jnp.bfloat16),
        ],
        compiler_params=pltpu.CompilerParams(
            dimension_semantics=("arbitrary",),
        ),
    )(x, W4, pmat, b2)
    return (out[0], out[1])


# speedup vs baseline: 1.0469x; 1.0469x over previous
import numpy as np
import jax
import jax.numpy as jnp
from jax.experimental import pallas as pl
from jax.experimental.pallas import tpu as pltpu

H = 384
W_IMG = 384
C = 3
P = 16
EMBED = 192
EPAD = 256
N_TARGETS = 4
HP = H // P
WP = W_IMG // P
NPATCH = HP * WP
LB = W_IMG // 128
WB = 128 // P


def _rect_indices(rng, h, w, sfr, arr):
    low_w = int(w * sfr[0])
    high_w = int(w * sfr[1])
    rec_width = int(rng.integers(max(low_w, 1), high_w + 1))
    low_h = int(rec_width * arr[0])
    high_h = int(rec_width * arr[1])
    rec_height = int(rng.integers(max(low_h, 1), high_h + 1))
    rec_height = min(rec_height, h)
    start_w = int(rng.integers(0, w - rec_width + 1))
    start_h = int(rng.integers(0, h - rec_height + 1))
    start = start_h * w + start_w
    idx = np.concatenate(
        [np.arange(start + i * w, start + i * w + rec_width) for i in range(rec_height)]
    )
    return idx.astype(np.int64)


def _mask_indices():
    rng = np.random.default_rng(0)
    targets = [
        _rect_indices(rng, HP, WP, (0.15, 0.2), (0.75, 1.5)) for _ in range(N_TARGETS)
    ]
    ctx = _rect_indices(rng, HP, WP, (0.85, 1.0), (1.0, 1.0))
    all_t = np.concatenate(targets)
    ctx = ctx[~np.isin(ctx, all_t)]
    return ctx, np.concatenate(targets)


CTX_IDX, TGT_IDX = _mask_indices()
N_CTX = len(CTX_IDX)
N_TGT = len(TGT_IDX)


def _runs(idx):
    runs = []
    start = 0
    for i in range(1, len(idx) + 1):
        if i == len(idx) or idx[i] != idx[i - 1] + 1:
            runs.append((int(idx[start]), start, i - start))
            start = i
    return runs


CTX_RUNS = _runs(CTX_IDX)
TGT_RUNS = _runs(TGT_IDX)


BN = 4


def _fused_kernel(x_ref, w4_ref, pm_ref, b_ref, ctx_ref, tgt_ref, wd_ref):
    n = pl.program_id(0)

    @pl.when(n == 0)
    def _init_weights():
        wd_ref[...] = jnp.zeros((P * C * 128, WB * EPAD), jnp.bfloat16)
        for p1 in range(P):
            for c in range(C):
                for w2 in range(WB):
                    r0 = (c * P + p1) * 128 + w2 * P
                    c0 = w2 * EPAD
                    wd_ref[r0 : r0 + P, c0 : c0 + EPAD] = w4_ref[c * P + p1]

    row_blocks = []
    for i in range(BN):
        pieces = []
        for c in range(C):
            xc = x_ref[i, c].reshape(H, W_IMG).astype(jnp.bfloat16)
            y = jax.lax.dot_general(
                pm_ref[...],
                xc,
                (((1,), (0,)), ((), ())),
                preferred_element_type=jnp.float32,
            ).astype(jnp.bfloat16)
            for p1 in range(P):
                blk = y[p1 * HP : (p1 + 1) * HP]
                pieces.append(blk.reshape(HP * LB, 128))
        row_blocks.append(jnp.concatenate(pieces, axis=1))
    lhs = jnp.concatenate(row_blocks, axis=0)
    acc = jax.lax.dot_general(
        lhs,
        wd_ref[...],
        (((1,), (0,)), ((), ())),
        preferred_element_type=jnp.float32,
    )
    for i in range(BN):
        emb = (
            acc[i * HP * LB : (i + 1) * HP * LB].reshape(NPATCH, EPAD)
            + b_ref[...]
        )
        for src, dst, ln in CTX_RUNS:
            ctx_ref[i, dst : dst + ln, :] = emb[src : src + ln, :EMBED]
        for src, dst, ln in TGT_RUNS:
            tgt_ref[i, dst : dst + ln, :] = emb[src : src + ln, :EMBED]


def kernel(x, W, b):
    B = x.shape[0]
    W4 = W.reshape(P, P, C, EMBED).transpose(2, 0, 1, 3).reshape(C * P, P, EMBED)
    W4 = jnp.pad(W4, ((0, 0), (0, 0), (0, EPAD - EMBED))).astype(jnp.bfloat16)
    b2 = jnp.pad(b, (0, EPAD - EMBED)).reshape(1, EPAD)
    pm = np.zeros((H, H), np.float32)
    for p1 in range(P):
        for h in range(HP):
            pm[p1 * HP + h, h * P + p1] = 1.0
    pmat = jnp.asarray(pm, dtype=jnp.bfloat16)

    grid = (B // BN,)
    out = pl.pallas_call(
        _fused_kernel,
        grid=grid,
        in_specs=[
            pl.BlockSpec((BN, C, H, W_IMG), lambda n: (n, 0, 0, 0)),
            pl.BlockSpec((C * P, P, EPAD), lambda n: (0, 0, 0)),
            pl.BlockSpec((H, H), lambda n: (0, 0)),
            pl.BlockSpec((1, EPAD), lambda n: (0, 0)),
        ],
        out_specs=[
            pl.BlockSpec((BN, N_CTX, EMBED), lambda n: (n, 0, 0)),
            pl.BlockSpec((BN, N_TGT, EMBED), lambda n: (n, 0, 0)),
        ],
        out_shape=[
            jax.ShapeDtypeStruct((B, N_CTX, EMBED), jnp.float32),
            jax.ShapeDtypeStruct((B, N_TGT, EMBED), jnp.float32),
        ],
        scratch_shapes=[
            pltpu.VMEM((P * C * 128, WB * EPAD), jnp.bfloat16),
        ],
        compiler_params=pltpu.CompilerParams(
            dimension_semantics=("arbitrary",),
            vmem_limit_bytes=110 * 1024 * 1024,
        ),
    )(x, W4, pmat, b2)
    return (out[0], out[1])


# speedup vs baseline: 1.0496x; 1.0026x over previous
import numpy as np
import jax
import jax.numpy as jnp
from jax.experimental import pallas as pl
from jax.experimental.pallas import tpu as pltpu

H = 384
W_IMG = 384
C = 3
P = 16
EMBED = 192
EPAD = 256
N_TARGETS = 4
HP = H // P
WP = W_IMG // P
NPATCH = HP * WP
LB = W_IMG // 128
WB = 128 // P


def _rect_indices(rng, h, w, sfr, arr):
    low_w = int(w * sfr[0])
    high_w = int(w * sfr[1])
    rec_width = int(rng.integers(max(low_w, 1), high_w + 1))
    low_h = int(rec_width * arr[0])
    high_h = int(rec_width * arr[1])
    rec_height = int(rng.integers(max(low_h, 1), high_h + 1))
    rec_height = min(rec_height, h)
    start_w = int(rng.integers(0, w - rec_width + 1))
    start_h = int(rng.integers(0, h - rec_height + 1))
    start = start_h * w + start_w
    idx = np.concatenate(
        [np.arange(start + i * w, start + i * w + rec_width) for i in range(rec_height)]
    )
    return idx.astype(np.int64)


def _mask_indices():
    rng = np.random.default_rng(0)
    targets = [
        _rect_indices(rng, HP, WP, (0.15, 0.2), (0.75, 1.5)) for _ in range(N_TARGETS)
    ]
    ctx = _rect_indices(rng, HP, WP, (0.85, 1.0), (1.0, 1.0))
    all_t = np.concatenate(targets)
    ctx = ctx[~np.isin(ctx, all_t)]
    return ctx, np.concatenate(targets)


CTX_IDX, TGT_IDX = _mask_indices()
N_CTX = len(CTX_IDX)
N_TGT = len(TGT_IDX)


def _runs(idx):
    runs = []
    start = 0
    for i in range(1, len(idx) + 1):
        if i == len(idx) or idx[i] != idx[i - 1] + 1:
            runs.append((int(idx[start]), start, i - start))
            start = i
    return runs


CTX_RUNS = _runs(CTX_IDX)
TGT_RUNS = _runs(TGT_IDX)


BN = 4


def _fused_kernel(x_ref, w4_ref, pm_ref, b_ref, ctx_ref, tgt_ref, wd_ref):
    n = pl.program_id(0)

    @pl.when(n == 0)
    def _init_weights():
        wd_ref[...] = jnp.zeros((P * C * 128, WB * EPAD), jnp.bfloat16)
        for p1 in range(P):
            for c in range(C):
                for w2 in range(WB):
                    r0 = (c * P + p1) * 128 + w2 * P
                    c0 = w2 * EPAD
                    wd_ref[r0 : r0 + P, c0 : c0 + EPAD] = w4_ref[c * P + p1]

    row_blocks = []
    for i in range(BN):
        pieces = []
        for c in range(C):
            xc = x_ref[i, c].reshape(H, W_IMG).astype(jnp.bfloat16)
            y = jax.lax.dot_general(
                pm_ref[...],
                xc,
                (((1,), (0,)), ((), ())),
                preferred_element_type=jnp.float32,
            ).astype(jnp.bfloat16)
            for p1 in range(P):
                blk = y[p1 * HP : (p1 + 1) * HP]
                pieces.append(blk.reshape(HP * LB, 128))
        row_blocks.append(jnp.concatenate(pieces, axis=1))
    lhs = jnp.concatenate(row_blocks, axis=0)
    acc = jax.lax.dot_general(
        lhs,
        wd_ref[...],
        (((1,), (0,)), ((), ())),
        preferred_element_type=jnp.float32,
    )
    for i in range(BN):
        emb = (
            acc[i * HP * LB : (i + 1) * HP * LB].reshape(NPATCH, EPAD)
            + b_ref[...]
        )
        for src, dst, ln in CTX_RUNS:
            ctx_ref[i, dst : dst + ln, :] = emb[src : src + ln, :EMBED]
        for src, dst, ln in TGT_RUNS:
            tgt_ref[i, dst : dst + ln, :] = emb[src : src + ln, :EMBED]


def kernel(x, W, b):
    B = x.shape[0]
    W4 = W.reshape(P, P, C, EMBED).transpose(2, 0, 1, 3).reshape(C * P, P, EMBED)
    W4 = jnp.pad(W4, ((0, 0), (0, 0), (0, EPAD - EMBED))).astype(jnp.bfloat16)
    b2 = jnp.pad(b, (0, EPAD - EMBED)).reshape(1, EPAD)
    pm = np.zeros((H, H), np.float32)
    for p1 in range(P):
        for h in range(HP):
            pm[p1 * HP + h, h * P + p1] = 1.0
    pmat = jnp.asarray(pm, dtype=jnp.bfloat16)

    grid = (B // BN,)
    out = pl.pallas_call(
        _fused_kernel,
        grid=grid,
        in_specs=[
            pl.BlockSpec((BN, C, H, W_IMG), lambda n: (n, 0, 0, 0)),
            pl.BlockSpec((C * P, P, EPAD), lambda n: (0, 0, 0)),
            pl.BlockSpec((H, H), lambda n: (0, 0)),
            pl.BlockSpec((1, EPAD), lambda n: (0, 0)),
        ],
        out_specs=[
            pl.BlockSpec((BN, N_CTX, EMBED), lambda n: (n, 0, 0)),
            pl.BlockSpec((BN, N_TGT, EMBED), lambda n: (n, 0, 0)),
        ],
        out_shape=[
            jax.ShapeDtypeStruct((B, N_CTX, EMBED), jnp.float32),
            jax.ShapeDtypeStruct((B, N_TGT, EMBED), jnp.float32),
        ],
        scratch_shapes=[
            pltpu.VMEM((P * C * 128, WB * EPAD), jnp.bfloat16),
        ],
    )(x, W4, pmat, b2)
    return (out[0], out[1])
